# Initial kernel scaffold; baseline (speedup 1.0000x reference)
import jax
import jax.numpy as jnp
from jax.experimental import pallas as pl


def kernel(panoptic_text, instance_text, semantic_text, pan_table, inst_table, sem_table, W, b):
    raise NotImplementedError("write your pallas kernel here")


# speedup vs baseline: 7.7885x; 7.7885x over previous
import functools

import jax
import jax.numpy as jnp
from jax import lax
from jax.experimental import pallas as pl
from jax.experimental.pallas import tpu as pltpu
from jax.experimental.pallas import tpu_sc as plsc

VOCAB = 100000
DIM = 64
B = 4096
L = 200

_NCHUNK = DIM // 16


def _sc_sums(pan_idx, inst_idx, sem_idx, pan_tab, inst_tab, sem_tab):
    info = plsc.get_sparse_core_info()
    nc, ns = info.num_cores, info.num_subcores
    nw = nc * ns
    rows_per_w = B // nw

    mesh = plsc.VectorSubcoreMesh(core_axis_name="c", subcore_axis_name="s")

    @functools.partial(
        pl.kernel,
        mesh=mesh,
        compiler_params=pltpu.CompilerParams(use_tc_tiling_on_sc=False),
        out_type=jax.ShapeDtypeStruct((3, B, DIM), jnp.float32),
        scratch_types=[
            pltpu.VMEM((L,), jnp.int32),
            pltpu.VMEM((L, DIM), jnp.float32),
            pltpu.VMEM((rows_per_w, DIM), jnp.float32),
            pltpu.SemaphoreType.DMA,
        ],
    )
    def sums_kernel(pan_i, inst_i, sem_i, pan_t, inst_t, sem_t, out_hbm,
                    idx_v, rows_v, sums_v, sem):
        wid = lax.axis_index("s") * nc + lax.axis_index("c")
        base = wid * rows_per_w

        for t, (idx_hbm, tab_hbm) in enumerate(
                ((pan_i, pan_t), (inst_i, inst_t), (sem_i, sem_t))):

            def row_body(i, _, idx_hbm=idx_hbm, tab_hbm=tab_hbm):
                b_row = base + i
                pltpu.sync_copy(idx_hbm.at[b_row], idx_v)
                cp0 = pltpu.async_copy(
                    tab_hbm.at[idx_v.at[pl.ds(0, 128)]],
                    rows_v.at[pl.ds(0, 128), :], sem)
                cp1 = pltpu.async_copy(
                    tab_hbm.at[idx_v.at[pl.ds(128, L - 128)]],
                    rows_v.at[pl.ds(128, L - 128), :], sem)
                cp0.wait()
                cp1.wait()

                def acc_body(r, accs):
                    return tuple(
                        accs[j] + rows_v[r, pl.ds(16 * j, 16)]
                        for j in range(_NCHUNK))

                zero = jnp.zeros((16,), jnp.float32)
                accs = lax.fori_loop(0, L, acc_body, (zero,) * _NCHUNK)
                for j in range(_NCHUNK):
                    sums_v[i, pl.ds(16 * j, 16)] = accs[j]
                return 0

            lax.fori_loop(0, rows_per_w, row_body, 0)
            pltpu.sync_copy(sums_v, out_hbm.at[t, pl.ds(base, rows_per_w), :])

    return sums_kernel(pan_idx, inst_idx, sem_idx, pan_tab, inst_tab, sem_tab)


def _proj_body(s_ref, wt_ref, b3_ref, o_ref):
    wt = wt_ref[...]
    outs = [
        jnp.dot(s_ref[t], wt, preferred_element_type=jnp.float32)
        for t in range(3)
    ]
    o_ref[...] = jnp.concatenate(outs, axis=-1) + b3_ref[...]


def _proj(sums, wt, b3):
    blk = 512
    return pl.pallas_call(
        _proj_body,
        grid=(B // blk,),
        in_specs=[
            pl.BlockSpec((3, blk, DIM), lambda i: (0, i, 0)),
            pl.BlockSpec((DIM, DIM), lambda i: (0, 0)),
            pl.BlockSpec((1, 3 * DIM), lambda i: (0, 0)),
        ],
        out_specs=pl.BlockSpec((blk, 3 * DIM), lambda i: (i, 0)),
        out_shape=jax.ShapeDtypeStruct((B, 3 * DIM), jnp.float32),
    )(sums, wt, b3)


def kernel(panoptic_text, instance_text, semantic_text, pan_table, inst_table,
           sem_table, W, b):
    pan_idx = panoptic_text.astype(jnp.int32)
    inst_idx = instance_text.astype(jnp.int32)
    sem_idx = semantic_text.astype(jnp.int32)

    sums = _sc_sums(pan_idx, inst_idx, sem_idx, pan_table, inst_table,
                    sem_table)

    wt = (W.T / jnp.float32(L)).astype(jnp.float32)
    b3 = jnp.tile(b, 3).reshape(1, 3 * DIM).astype(jnp.float32)
    out2d = _proj(sums, wt, b3)
    return out2d.reshape(B, 3, DIM)


# speedup vs baseline: 15.6189x; 2.0054x over previous
import functools

import jax
import jax.numpy as jnp
from jax import lax
from jax.experimental import pallas as pl
from jax.experimental.pallas import tpu as pltpu
from jax.experimental.pallas import tpu_sc as plsc

VOCAB = 100000
DIM = 64
B = 4096
L = 200

_NCHUNK = DIM // 16


def _sc_sums(pan_idx, inst_idx, sem_idx, pan_tab, inst_tab, sem_tab):
    info = plsc.get_sparse_core_info()
    nc, ns = info.num_cores, info.num_subcores
    nw = nc * ns
    rows_per_w = B // nw

    mesh = plsc.VectorSubcoreMesh(core_axis_name="c", subcore_axis_name="s")

    @functools.partial(
        pl.kernel,
        mesh=mesh,
        compiler_params=pltpu.CompilerParams(use_tc_tiling_on_sc=False),
        out_type=jax.ShapeDtypeStruct((3, B, DIM), jnp.float32),
        scratch_types=[
            pltpu.VMEM((rows_per_w, L), jnp.int32),
            pltpu.VMEM((L, DIM), jnp.float32),
            pltpu.VMEM((L, DIM), jnp.float32),
            pltpu.VMEM((rows_per_w, DIM), jnp.float32),
            pltpu.SemaphoreType.DMA,
            pltpu.SemaphoreType.DMA,
        ],
    )
    def sums_kernel(pan_i, inst_i, sem_i, pan_t, inst_t, sem_t, out_hbm,
                    idx_all, buf0, buf1, sums_v, sem0, sem1):
        wid = lax.axis_index("s") * nc + lax.axis_index("c")
        base = wid * rows_per_w
        bufs = (buf0, buf1)
        sems = (sem0, sem1)

        for t, (idx_hbm, tab_hbm) in enumerate(
                ((pan_i, pan_t), (inst_i, inst_t), (sem_i, sem_t))):

            pltpu.sync_copy(idx_hbm.at[pl.ds(base, rows_per_w)], idx_all)

            def start_gather(i, slot, tab_hbm=tab_hbm):
                pltpu.async_copy(
                    tab_hbm.at[idx_all.at[i, pl.ds(0, 128)]],
                    bufs[slot].at[pl.ds(0, 128), :], sems[slot])
                pltpu.async_copy(
                    tab_hbm.at[idx_all.at[i, pl.ds(128, L - 128)]],
                    bufs[slot].at[pl.ds(128, L - 128), :], sems[slot])

            start_gather(0, 0)

            def pair_body(g, _, tab_hbm=tab_hbm):
                for s in range(2):
                    i = 2 * g + s

                    @pl.when(i + 1 < rows_per_w)
                    def _(i=i, s=s):
                        start_gather(i + 1, (s + 1) % 2)

                    pltpu.make_async_copy(
                        tab_hbm.at[pl.ds(0, L), :], bufs[s], sems[s]).wait()

                    buf = bufs[s]

                    def acc_body(r, accs, buf=buf):
                        return tuple(
                            accs[j] + buf[r, pl.ds(16 * j, 16)]
                            for j in range(_NCHUNK))

                    zero = jnp.zeros((16,), jnp.float32)
                    accs = lax.fori_loop(0, L, acc_body, (zero,) * _NCHUNK,
                                         unroll=8)
                    for j in range(_NCHUNK):
                        sums_v[i, pl.ds(16 * j, 16)] = accs[j]
                return 0

            lax.fori_loop(0, rows_per_w // 2, pair_body, 0)
            pltpu.sync_copy(sums_v, out_hbm.at[t, pl.ds(base, rows_per_w), :])

    return sums_kernel(pan_idx, inst_idx, sem_idx, pan_tab, inst_tab, sem_tab)


def _proj_body(s_ref, wt_ref, b3_ref, o_ref):
    wt = wt_ref[...]
    outs = [
        jnp.dot(s_ref[t], wt, preferred_element_type=jnp.float32)
        for t in range(3)
    ]
    o_ref[...] = jnp.concatenate(outs, axis=-1) + b3_ref[...]


def _proj(sums, wt, b3):
    blk = 512
    return pl.pallas_call(
        _proj_body,
        grid=(B // blk,),
        in_specs=[
            pl.BlockSpec((3, blk, DIM), lambda i: (0, i, 0)),
            pl.BlockSpec((DIM, DIM), lambda i: (0, 0)),
            pl.BlockSpec((1, 3 * DIM), lambda i: (0, 0)),
        ],
        out_specs=pl.BlockSpec((blk, 3 * DIM), lambda i: (i, 0)),
        out_shape=jax.ShapeDtypeStruct((B, 3 * DIM), jnp.float32),
    )(sums, wt, b3)


def kernel(panoptic_text, instance_text, semantic_text, pan_table, inst_table,
           sem_table, W, b):
    pan_idx = panoptic_text.astype(jnp.int32)
    inst_idx = instance_text.astype(jnp.int32)
    sem_idx = semantic_text.astype(jnp.int32)

    sums = _sc_sums(pan_idx, inst_idx, sem_idx, pan_table, inst_table,
                    sem_table)

    wt = (W.T / jnp.float32(L)).astype(jnp.float32)
    b3 = jnp.tile(b, 3).reshape(1, 3 * DIM).astype(jnp.float32)
    out2d = _proj(sums, wt, b3)
    return out2d.reshape(B, 3, DIM)


# speedup vs baseline: 19.7485x; 1.2644x over previous
import functools

import jax
import jax.numpy as jnp
from jax import lax
from jax.experimental import pallas as pl
from jax.experimental.pallas import tpu as pltpu
from jax.experimental.pallas import tpu_sc as plsc

VOCAB = 100000
DIM = 64
B = 4096
L = 200

_NCHUNK = DIM // 16


def _sc_sums(pan_idx, inst_idx, sem_idx, pan_tab, inst_tab, sem_tab):
    info = plsc.get_sparse_core_info()
    nc, ns = info.num_cores, info.num_subcores
    nw = nc * ns
    rows_per_w = B // nw

    mesh = plsc.VectorSubcoreMesh(core_axis_name="c", subcore_axis_name="s")

    @functools.partial(
        pl.kernel,
        mesh=mesh,
        compiler_params=pltpu.CompilerParams(use_tc_tiling_on_sc=False),
        out_type=jax.ShapeDtypeStruct((3, B, DIM), jnp.float32),
        scratch_types=[
            pltpu.VMEM((rows_per_w, L), jnp.int32),
            pltpu.VMEM((L, DIM), jnp.float32),
            pltpu.VMEM((L, DIM), jnp.float32),
            pltpu.VMEM((L, DIM), jnp.float32),
            pltpu.VMEM((L, DIM), jnp.float32),
            pltpu.VMEM((rows_per_w, DIM), jnp.float32),
            pltpu.SemaphoreType.DMA,
            pltpu.SemaphoreType.DMA,
            pltpu.SemaphoreType.DMA,
            pltpu.SemaphoreType.DMA,
        ],
    )
    def sums_kernel(pan_i, inst_i, sem_i, pan_t, inst_t, sem_t, out_hbm,
                    idx_all, buf0, buf1, buf2, buf3, sums_v,
                    sem0, sem1, sem2, sem3):
        wid = lax.axis_index("s") * nc + lax.axis_index("c")
        base = wid * rows_per_w
        bufs = (buf0, buf1, buf2, buf3)
        sems = (sem0, sem1, sem2, sem3)
        nbuf = 4

        for t, (idx_hbm, tab_hbm) in enumerate(
                ((pan_i, pan_t), (inst_i, inst_t), (sem_i, sem_t))):

            pltpu.sync_copy(idx_hbm.at[pl.ds(base, rows_per_w)], idx_all)

            def start_gather(i, slot, tab_hbm=tab_hbm):
                pltpu.async_copy(
                    tab_hbm.at[idx_all.at[i, pl.ds(0, 128)]],
                    bufs[slot].at[pl.ds(0, 128), :], sems[slot])
                pltpu.async_copy(
                    tab_hbm.at[idx_all.at[i, pl.ds(128, L - 128)]],
                    bufs[slot].at[pl.ds(128, L - 128), :], sems[slot])

            for p in range(nbuf - 1):
                start_gather(p, p)

            def pair_body(g, _, tab_hbm=tab_hbm):
                for s in range(nbuf):
                    i = nbuf * g + s

                    @pl.when(i + nbuf - 1 < rows_per_w)
                    def _(i=i, s=s):
                        start_gather(i + nbuf - 1, (s + nbuf - 1) % nbuf)

                    pltpu.make_async_copy(
                        tab_hbm.at[pl.ds(0, L), :], bufs[s], sems[s]).wait()

                    buf = bufs[s]

                    def acc_body(r, accs, buf=buf):
                        return tuple(
                            accs[j] + buf[r, pl.ds(16 * j, 16)]
                            for j in range(_NCHUNK))

                    zero = jnp.zeros((16,), jnp.float32)
                    accs = lax.fori_loop(0, L, acc_body, (zero,) * _NCHUNK,
                                         unroll=8)
                    for j in range(_NCHUNK):
                        sums_v[i, pl.ds(16 * j, 16)] = accs[j]
                return 0

            lax.fori_loop(0, rows_per_w // nbuf, pair_body, 0)
            pltpu.sync_copy(sums_v, out_hbm.at[t, pl.ds(base, rows_per_w), :])

    return sums_kernel(pan_idx, inst_idx, sem_idx, pan_tab, inst_tab, sem_tab)


def _proj_body(s_ref, wt_ref, b3_ref, o_ref):
    wt = wt_ref[...]
    outs = [
        jnp.dot(s_ref[t], wt, preferred_element_type=jnp.float32)
        for t in range(3)
    ]
    o_ref[...] = jnp.concatenate(outs, axis=-1) + b3_ref[...]


def _proj(sums, wt, b3):
    blk = 512
    return pl.pallas_call(
        _proj_body,
        grid=(B // blk,),
        in_specs=[
            pl.BlockSpec((3, blk, DIM), lambda i: (0, i, 0)),
            pl.BlockSpec((DIM, DIM), lambda i: (0, 0)),
            pl.BlockSpec((1, 3 * DIM), lambda i: (0, 0)),
        ],
        out_specs=pl.BlockSpec((blk, 3 * DIM), lambda i: (i, 0)),
        out_shape=jax.ShapeDtypeStruct((B, 3 * DIM), jnp.float32),
    )(sums, wt, b3)


def kernel(panoptic_text, instance_text, semantic_text, pan_table, inst_table,
           sem_table, W, b):
    pan_idx = panoptic_text.astype(jnp.int32)
    inst_idx = instance_text.astype(jnp.int32)
    sem_idx = semantic_text.astype(jnp.int32)

    sums = _sc_sums(pan_idx, inst_idx, sem_idx, pan_table, inst_table,
                    sem_table)

    wt = (W.T / jnp.float32(L)).astype(jnp.float32)
    b3 = jnp.tile(b, 3).reshape(1, 3 * DIM).astype(jnp.float32)
    out2d = _proj(sums, wt, b3)
    return out2d.reshape(B, 3, DIM)
